# baseline (speedup 1.0000x reference)
import functools

import jax
import jax.numpy as jnp
from jax import lax
from jax.experimental import pallas as pl
from jax.experimental.pallas import tpu as pltpu
from jax.experimental.pallas import tpu_sc as plsc

N = 10000
E = 320000
NT = 16
E_T = E // NT
CH = 80
NCH = E_T // CH
RT = 640
ACC_R = RT * NT
RT_LAST = N - RT * (NT - 1)

_MESH = plsc.VectorSubcoreMesh(core_axis_name="c", subcore_axis_name="s")
_f32 = jnp.float32
R = 5
NBLK = 5
BCH = NCH // NBLK


def _pipe_gather_scatter(xs, s4, d4, accs, sid, sidxb, didxb, lane_rows,
                         lane_gsems, lane_ssems):
    NL = len(xs)
    for blk in range(NBLK):
        pltpu.sync_copy(s4.at[sid * NBLK + blk], sidxb)
        pltpu.sync_copy(d4.at[sid * NBLK + blk], didxb)
        for L in range(NL):
            pltpu.async_copy(xs[L].at[sidxb.at[0]], lane_rows[L][0],
                             lane_gsems[L][0])

        def round_body(r, carry):
            for k in range(2):
                b = k
                bo = 1 - k
                c = r * 2 + k
                for L in range(NL):
                    pltpu.make_async_copy(xs[L].at[sidxb.at[0]],
                                          lane_rows[L][b],
                                          lane_gsems[L][b]).wait()
                for L in range(NL):
                    if k == 0:
                        @pl.when(r > 0)
                        def _(L=L):
                            pltpu.make_async_copy(
                                lane_rows[L][bo], accs[L].at[didxb.at[0]],
                                lane_ssems[L][bo]).wait()
                    else:
                        pltpu.make_async_copy(
                            lane_rows[L][bo], accs[L].at[didxb.at[0]],
                            lane_ssems[L][bo]).wait()

                for L in range(NL):
                    @pl.when(c + 1 < BCH)
                    def _(c=c, bo=bo, L=L):
                        pltpu.async_copy(xs[L].at[sidxb.at[c + 1]],
                                         lane_rows[L][bo], lane_gsems[L][bo])

                for L in range(NL):
                    pltpu.async_copy(lane_rows[L][b], accs[L].at[didxb.at[c]],
                                     lane_ssems[L][b], add=True)
            return carry

        lax.fori_loop(0, BCH // 2, round_body, 0)
        for L in range(NL):
            pltpu.make_async_copy(lane_rows[L][1], accs[L].at[didxb.at[0]],
                                  lane_ssems[L][1]).wait()


def _pipe_gather_scatter_par(x, s4, d4, accs2, sid, sidxb, didxb, rows4,
                             gsems4, ssems4):
    NB4 = (BCH // 4) * 4
    for blk in range(NBLK):
        pltpu.sync_copy(s4.at[sid * NBLK + blk], sidxb)
        pltpu.sync_copy(d4.at[sid * NBLK + blk], didxb)
        pltpu.async_copy(x.at[sidxb.at[0]], rows4[0], gsems4[0])
        pltpu.async_copy(x.at[sidxb.at[1]], rows4[1], gsems4[1])

        def round_body(r, carry):
            for k in range(4):
                b = k
                b2 = (k + 2) % 4
                c = r * 4 + k
                pltpu.make_async_copy(x.at[sidxb.at[0]], rows4[b],
                                      gsems4[b]).wait()
                if k < 2:
                    @pl.when(r > 0)
                    def _(b2=b2):
                        pltpu.make_async_copy(rows4[b2],
                                              accs2[b2 % 2].at[didxb.at[0]],
                                              ssems4[b2]).wait()
                else:
                    pltpu.make_async_copy(rows4[b2],
                                          accs2[b2 % 2].at[didxb.at[0]],
                                          ssems4[b2]).wait()
                pltpu.async_copy(x.at[sidxb.at[c + 2]], rows4[b2],
                                 gsems4[b2])
                pltpu.async_copy(rows4[b], accs2[b % 2].at[didxb.at[c]],
                                 ssems4[b], add=True)
            return carry

        lax.fori_loop(0, NB4 // 4, round_body, 0)
        for c in range(NB4, BCH):
            b = c % 4
            b2 = (c + 2) % 4
            pltpu.make_async_copy(x.at[sidxb.at[0]], rows4[b],
                                  gsems4[b]).wait()
            pltpu.make_async_copy(rows4[b2], accs2[b2 % 2].at[didxb.at[0]],
                                  ssems4[b2]).wait()
            pltpu.async_copy(rows4[b], accs2[b % 2].at[didxb.at[c]],
                             ssems4[b], add=True)
        for c in range(BCH - 2, BCH):
            b = c % 4
            pltpu.make_async_copy(rows4[b], accs2[b % 2].at[didxb.at[0]],
                                  ssems4[b]).wait()


def _pipe_hist(arr3, hists, sid, idx_all, onesv, ssems):
    pltpu.sync_copy(arr3.at[sid], idx_all)

    def round_body(r, carry):
        for k in range(R):
            c = r * R + k

            @pl.when(r > 0)
            def _(k=k):
                pltpu.make_async_copy(onesv, hists[k].at[idx_all.at[0]],
                                      ssems[k]).wait()

            pltpu.async_copy(onesv, hists[k].at[idx_all.at[c]], ssems[k],
                             add=True)
        return carry

    lax.fori_loop(0, NCH // R, round_body, 0)
    for k in range(R):
        pltpu.make_async_copy(onesv, hists[k].at[idx_all.at[0]],
                              ssems[k]).wait()


def _reduce_hists(hists, out, sid, rbuf, obuf, sz):
    off = sid * 640
    for j in range(R):
        pltpu.sync_copy(hists[j].at[pl.ds(off, sz)],
                        rbuf.at[j, pl.ds(0, sz)])
    for i in range(sz // 16):
        v = rbuf[0, pl.ds(16 * i, 16)]
        for j in range(1, R):
            v = v + rbuf[j, pl.ds(16 * i, 16)]
        obuf[pl.ds(16 * i, 16)] = v
    pltpu.sync_copy(obuf.at[pl.ds(0, sz)], out.at[pl.ds(off, sz)])


def _deg_kernel(src0, dst0, src1, dst1, ones_hbm, zvec):
    @functools.partial(
        pl.kernel,
        out_type=[jax.ShapeDtypeStruct((N,), _f32) for _ in range(4)],
        mesh=_MESH,
        scratch_types=[
            pltpu.VMEM((NCH, CH), jnp.int32),
            pltpu.VMEM((CH,), _f32),
            pltpu.VMEM((R, 640), _f32),
            pltpu.VMEM((640,), _f32),
        ] + [pltpu.VMEM_SHARED((N,), _f32)] * (2 * R)
          + [pltpu.SemaphoreType.DMA] * R,
        compiler_params=pltpu.CompilerParams(use_tc_tiling_on_sc=False),
    )
    def k(s0, d0, s1, d1, ones_h, zv, o0, o1, o2, o3, idx_all, onesv,
          rbuf, obuf, *bufs):
        hista = bufs[:R]
        histb = bufs[R:2 * R]
        ssems = bufs[2 * R:]
        sid = lax.axis_index("s")
        cid = lax.axis_index("c")
        pltpu.sync_copy(ones_h, onesv)

        @pl.when(sid == 0)
        def _():
            for h in hista + histb:
                pltpu.sync_copy(zv, h)

        plsc.subcore_barrier()

        @pl.when(cid == 0)
        def _():
            _pipe_hist(s0, hista, sid, idx_all, onesv, ssems)
            _pipe_hist(d0, histb, sid, idx_all, onesv, ssems)

        @pl.when(cid == 1)
        def _():
            _pipe_hist(s1, hista, sid, idx_all, onesv, ssems)
            _pipe_hist(d1, histb, sid, idx_all, onesv, ssems)

        plsc.subcore_barrier()

        for c, (oa, ob) in enumerate([(o0, o1), (o2, o3)]):
            @pl.when(jnp.logical_and(cid == c, sid < NT - 1))
            def _(oa=oa, ob=ob):
                _reduce_hists(hista, oa, sid, rbuf, obuf, 640)
                _reduce_hists(histb, ob, sid, rbuf, obuf, 640)

            @pl.when(jnp.logical_and(cid == c, sid == NT - 1))
            def _(oa=oa, ob=ob):
                _reduce_hists(hista, oa, sid, rbuf, obuf, 400)
                _reduce_hists(histb, ob, sid, rbuf, obuf, 400)

    return k(src0, dst0, src1, dst1, ones_hbm, zvec)


def _make_agg(D):
    H = D // 2

    @functools.partial(
        pl.kernel,
        out_type=[jax.ShapeDtypeStruct((N, H), _f32) for _ in range(4)],
        mesh=_MESH,
        scratch_types=[
            pltpu.VMEM((BCH, CH), jnp.int32),
            pltpu.VMEM((BCH, CH), jnp.int32),
            pltpu.VMEM_SHARED((ACC_R, H), _f32),
            pltpu.VMEM_SHARED((ACC_R, H), _f32),
        ] + [pltpu.VMEM((CH, H), _f32)] * 4
          + [pltpu.SemaphoreType.DMA] * 8,
        compiler_params=pltpu.CompilerParams(use_tc_tiling_on_sc=False),
    )
    def k(x0_lo, x0_hi, s0, d0, x1_lo, x1_hi, s1, d1, zrow,
          o0_lo, o0_hi, o1_lo, o1_hi, sidx_all, didx_all,
          acc_lo, acc_hi, *bufs):
        lane_rows = [bufs[0:2], bufs[2:4]]
        lane_gsems = [bufs[4:6], bufs[6:8]]
        lane_ssems = [bufs[8:10], bufs[10:12]]
        accs = (acc_lo, acc_hi)
        sid = lax.axis_index("s")
        cid = lax.axis_index("c")
        pltpu.sync_copy(zrow, acc_lo.at[pl.ds(sid * RT, RT)])
        pltpu.sync_copy(zrow, acc_hi.at[pl.ds(sid * RT, RT)])
        plsc.subcore_barrier()

        @pl.when(cid == 0)
        def _():
            _pipe_gather_scatter((x0_lo, x0_hi), s0, d0, accs, sid,
                                 sidx_all, didx_all, lane_rows,
                                 lane_gsems, lane_ssems)

        @pl.when(cid == 1)
        def _():
            _pipe_gather_scatter((x1_lo, x1_hi), s1, d1, accs, sid,
                                 sidx_all, didx_all, lane_rows,
                                 lane_gsems, lane_ssems)

        plsc.subcore_barrier()

        outs = [(o0_lo, o0_hi), (o1_lo, o1_hi)]
        for c in range(2):
            @pl.when(jnp.logical_and(cid == c, sid < NT - 1))
            def _(c=c):
                sl = pl.ds(sid * RT, RT)
                pltpu.sync_copy(acc_lo.at[sl], outs[c][0].at[sl])
                pltpu.sync_copy(acc_hi.at[sl], outs[c][1].at[sl])

            @pl.when(jnp.logical_and(cid == c, sid == NT - 1))
            def _(c=c):
                sl = pl.ds((NT - 1) * RT, RT_LAST)
                pltpu.sync_copy(acc_lo.at[sl], outs[c][0].at[sl])
                pltpu.sync_copy(acc_hi.at[sl], outs[c][1].at[sl])

    return k


_agg128 = _make_agg(128)


def _agg_l3(x32_lo, x32_hi, s0, d0, xe, s1, d1, zrow32, zvec):
    @functools.partial(
        pl.kernel,
        out_type=[jax.ShapeDtypeStruct((N, 32), _f32),
                  jax.ShapeDtypeStruct((N, 32), _f32),
                  jax.ShapeDtypeStruct((N,), _f32),
                  jax.ShapeDtypeStruct((N,), _f32)],
        mesh=_MESH,
        scratch_types=[
            pltpu.VMEM((BCH, CH), jnp.int32),
            pltpu.VMEM((BCH, CH), jnp.int32),
            pltpu.VMEM_SHARED((ACC_R, 32), _f32),
            pltpu.VMEM_SHARED((ACC_R, 32), _f32),
            pltpu.VMEM_SHARED((N,), _f32),
            pltpu.VMEM_SHARED((N,), _f32),
        ] + [pltpu.VMEM((CH, 32), _f32)] * 4
          + [pltpu.VMEM((CH,), _f32)] * 4
          + [pltpu.SemaphoreType.DMA] * 8,
        compiler_params=pltpu.CompilerParams(use_tc_tiling_on_sc=False),
    )
    def k(x_lo, x_hi, s_0, d_0, x_e, s_1, d_1, zr, zv,
          o32_lo, o32_hi, oe_a, oe_b, sidx_all, didx_all,
          acc_lo, acc_hi, acc1a, acc1b, *bufs):
        lane_rows = [bufs[0:2], bufs[2:4]]
        vals4 = bufs[4:8]
        sems_a = bufs[8:12]
        sems_b = bufs[12:16]
        sid = lax.axis_index("s")
        cid = lax.axis_index("c")

        @pl.when(cid == 0)
        def _():
            pltpu.sync_copy(zr, acc_lo.at[pl.ds(sid * RT, RT)])
            pltpu.sync_copy(zr, acc_hi.at[pl.ds(sid * RT, RT)])

        @pl.when(jnp.logical_and(cid == 1, sid == 0))
        def _():
            pltpu.sync_copy(zv, acc1a)
            pltpu.sync_copy(zv, acc1b)

        plsc.subcore_barrier()

        @pl.when(cid == 0)
        def _():
            _pipe_gather_scatter((x_lo, x_hi), s_0, d_0, (acc_lo, acc_hi),
                                 sid, sidx_all, didx_all, lane_rows,
                                 [sems_a[0:2], sems_a[2:4]],
                                 [sems_b[0:2], sems_b[2:4]])

        @pl.when(cid == 1)
        def _():
            _pipe_gather_scatter((x_e,), s_1, d_1, (acc1a,), sid,
                                 sidx_all, didx_all, [vals4[0:2]],
                                 [sems_a[0:2]], [sems_b[0:2]])

        plsc.subcore_barrier()

        @pl.when(jnp.logical_and(cid == 0, sid < NT - 1))
        def _():
            sl = pl.ds(sid * RT, RT)
            pltpu.sync_copy(acc_lo.at[sl], o32_lo.at[sl])
            pltpu.sync_copy(acc_hi.at[sl], o32_hi.at[sl])

        @pl.when(jnp.logical_and(cid == 0, sid == NT - 1))
        def _():
            sl = pl.ds((NT - 1) * RT, RT_LAST)
            pltpu.sync_copy(acc_lo.at[sl], o32_lo.at[sl])
            pltpu.sync_copy(acc_hi.at[sl], o32_hi.at[sl])

        @pl.when(jnp.logical_and(cid == 1, sid == 0))
        def _():
            pltpu.sync_copy(acc1a, oe_a)
            pltpu.sync_copy(acc1b, oe_b)

    return k(x32_lo, x32_hi, s0, d0, xe, s1, d1, zrow32, zvec)


_B = 1000
_G = N // _B


def _row_spec(d):
    return pl.BlockSpec((_B, d), lambda i: (i, 0))


def _full_spec(shape):
    if len(shape) == 1:
        return pl.BlockSpec(shape, lambda i: (0,))
    return pl.BlockSpec(shape, lambda i: (0, 0))


def _pre_body(x, emb, so_ti, so_it, Wi, bi, Wh, bh, Wo, bo,
              y0t_lo, y0t_hi, y0i_lo, y0i_hi):
    h = jax.nn.relu(x[...] @ Wi[...] + bi[...])
    h = jax.nn.relu(h @ Wh[...] + bh[...])
    t = (h @ Wo[...] + bo[...]) * so_ti[...]
    y0t_lo[...] = t[:, :64]
    y0t_hi[...] = t[:, 64:]
    e = emb[...] * so_it[...]
    y0i_lo[...] = e[:, :64]
    y0i_hi[...] = e[:, 64:]


def _pre(x, emb, so_ti, so_it, Wi, bi, Wh, bh, Wo, bo):
    return pl.pallas_call(
        _pre_body,
        grid=(_G,),
        in_specs=[_row_spec(256), _row_spec(128), _row_spec(1), _row_spec(1),
                  _full_spec(Wi.shape), _full_spec(bi.shape),
                  _full_spec(Wh.shape), _full_spec(bh.shape),
                  _full_spec(Wo.shape), _full_spec(bo.shape)],
        out_specs=[_row_spec(64)] * 4,
        out_shape=[jax.ShapeDtypeStruct((N, 64), _f32)] * 4,
    )(x, emb, so_ti, so_it, Wi, bi, Wh, bh, Wo, bo)


def _l1_body(at_lo, at_hi, ai_lo, ai_hi, si_t, si_i, so_it, so_ti,
             W_it_a, W_it_b, b_it, W_ti_a, W_ti_b, b_ti,
             y_i_lo, y_i_hi, y_t_lo, y_t_hi):
    h_t = jax.nn.relu((at_lo[...] * si_t[...]) @ W_it_a[...]
                      + (at_hi[...] * si_t[...]) @ W_it_b[...] + b_it[...])
    y_t = h_t * so_ti[...]
    y_t_lo[...] = y_t[:, :64]
    y_t_hi[...] = y_t[:, 64:]
    h_i = jax.nn.relu((ai_lo[...] * si_i[...]) @ W_ti_a[...]
                      + (ai_hi[...] * si_i[...]) @ W_ti_b[...] + b_ti[...])
    y_i = h_i * so_it[...]
    y_i_lo[...] = y_i[:, :64]
    y_i_hi[...] = y_i[:, 64:]


def _l1(at_lo, at_hi, ai_lo, ai_hi, si_t, si_i, so_it, so_ti,
        W_it, b_it, W_ti, b_ti):
    return pl.pallas_call(
        _l1_body,
        grid=(_G,),
        in_specs=[_row_spec(64)] * 4 + [_row_spec(1)] * 4 +
                 [_full_spec((64, 128)), _full_spec((64, 128)),
                  _full_spec(b_it.shape),
                  _full_spec((64, 128)), _full_spec((64, 128)),
                  _full_spec(b_ti.shape)],
        out_specs=[_row_spec(64)] * 4,
        out_shape=[jax.ShapeDtypeStruct((N, 64), _f32)] * 4,
    )(at_lo, at_hi, ai_lo, ai_hi, si_t, si_i, so_it, so_ti,
      W_it[:64], W_it[64:], b_it, W_ti[:64], W_ti[64:], b_ti)


def _l2_body(at_lo, at_hi, ai_lo, ai_hi, si_t, si_i, so_it, so_ti,
             W_it_a, W_it_b, b_it, W_ti_a, W_ti_b, b_ti,
             W3_it_a, W3_it_b, W3_ti,
             z_lo, z_hi, z_ti):
    h2_t = jax.nn.relu((at_lo[...] * si_t[...]) @ W_it_a[...]
                       + (at_hi[...] * si_t[...]) @ W_it_b[...] + b_it[...])
    h2_i = jax.nn.relu((ai_lo[...] * si_i[...]) @ W_ti_a[...]
                       + (ai_hi[...] * si_i[...]) @ W_ti_b[...] + b_ti[...])
    zi = h2_i * so_it[...]
    z_lo[...] = zi @ W3_it_a[...]
    z_hi[...] = zi @ W3_it_b[...]
    zt = h2_t * so_ti[...]
    z_ti[...] = zt @ W3_ti[...]


def _l2(at_lo, at_hi, ai_lo, ai_hi, si_t, si_i, so_it, so_ti,
        W_it, b_it, W_ti, b_ti, W3_it, W3_ti):
    return pl.pallas_call(
        _l2_body,
        grid=(_G,),
        in_specs=[_row_spec(64)] * 4 + [_row_spec(1)] * 4 +
                 [_full_spec((64, 128)), _full_spec((64, 128)),
                  _full_spec(b_it.shape),
                  _full_spec((64, 128)), _full_spec((64, 128)),
                  _full_spec(b_ti.shape),
                  _full_spec((128, 32)), _full_spec((128, 32)),
                  _full_spec((128, 1))],
        out_specs=[_row_spec(32), _row_spec(32), _row_spec(1)],
        out_shape=[jax.ShapeDtypeStruct((N, 32), _f32),
                   jax.ShapeDtypeStruct((N, 32), _f32),
                   jax.ShapeDtypeStruct((N, 1), _f32)],
    )(at_lo, at_hi, ai_lo, ai_hi, si_t, si_i, so_it, so_ti,
      W_it[:64], W_it[64:], b_it, W_ti[:64], W_ti[64:], b_ti,
      W3_it[:, :32], W3_it[:, 32:], W3_ti)


def _post_body(a32_lo, a32_hi, age_a, age_b, si_t, si_i,
               b3_lo, b3_hi, b3_ti, Wi_a, Wi_b, bi, Wh, bh, Wo, bo,
               o_t, o_i):
    t_lo = a32_lo[...] * si_t[...] + b3_lo[...]
    t_hi = a32_hi[...] * si_t[...] + b3_hi[...]
    h = jax.nn.relu(t_lo @ Wi_a[...] + t_hi @ Wi_b[...] + bi[...])
    h = jax.nn.relu(h @ Wh[...] + bh[...])
    o_t[...] = h @ Wo[...] + bo[...]
    o_i[...] = (age_a[...] + age_b[...]) * si_i[...] + b3_ti[...]


def _post(a32_lo, a32_hi, age_a, age_b, si_t, si_i, b3_it, b3_ti,
          Wi, bi, Wh, bh, Wo, bo):
    return pl.pallas_call(
        _post_body,
        grid=(_G,),
        in_specs=[_row_spec(32), _row_spec(32), _row_spec(1), _row_spec(1),
                  _row_spec(1), _row_spec(1),
                  _full_spec((32,)), _full_spec((32,)),
                  _full_spec(b3_ti.shape),
                  _full_spec((32, 64)), _full_spec((32, 64)),
                  _full_spec(bi.shape),
                  _full_spec(Wh.shape), _full_spec(bh.shape),
                  _full_spec(Wo.shape), _full_spec(bo.shape)],
        out_specs=[_row_spec(1), _row_spec(1)],
        out_shape=[jax.ShapeDtypeStruct((N, 1), _f32),
                   jax.ShapeDtypeStruct((N, 1), _f32)],
    )(a32_lo, a32_hi, age_a, age_b, si_t, si_i, b3_it[:32], b3_it[32:],
      b3_ti, Wi[:32], Wi[32:], bi, Wh, bh, Wo, bo)


def kernel(input_features, edge_i2t, edge_t2i, embed_item,
           pre_Wi, pre_bi, pre_Wh, pre_bh, pre_Wo, pre_bo,
           c1_W_i2t, c1_b_i2t, c1_W_t2i, c1_b_t2i,
           c2_W_i2t, c2_b_i2t, c2_W_t2i, c2_b_t2i,
           c3_W_i2t, c3_b_i2t, c3_W_t2i, c3_b_t2i,
           post_Wi, post_bi, post_Wh, post_bh, post_Wo, post_bo):
    src_it = edge_i2t[0].reshape(NT * NBLK, BCH, CH)
    dst_it = edge_i2t[1].reshape(NT * NBLK, BCH, CH)
    src_ti = edge_t2i[0].reshape(NT * NBLK, BCH, CH)
    dst_ti = edge_t2i[1].reshape(NT * NBLK, BCH, CH)
    src_it_t = edge_i2t[0].reshape(NT, NCH, CH)
    dst_it_t = edge_i2t[1].reshape(NT, NCH, CH)
    src_ti_t = edge_t2i[0].reshape(NT, NCH, CH)
    dst_ti_t = edge_t2i[1].reshape(NT, NCH, CH)

    ones_hbm = jnp.ones((CH,), _f32)
    zvec = jnp.zeros((N,), _f32)
    zrow64 = jnp.zeros((RT, 64), _f32)
    zrow32 = jnp.zeros((RT, 32), _f32)

    d_out_it, d_in_t, d_out_ti, d_in_i = _deg_kernel(
        src_it_t, dst_it_t, src_ti_t, dst_ti_t, ones_hbm, zvec)

    def scale(d):
        return (jnp.clip(d, 1.0, None) ** -0.5)[:, None]

    so_it, si_t, so_ti, si_i = map(scale, (d_out_it, d_in_t, d_out_ti, d_in_i))

    y0t_lo, y0t_hi, y0i_lo, y0i_hi = _pre(
        input_features.astype(_f32), embed_item, so_ti, so_it,
        pre_Wi, pre_bi, pre_Wh, pre_bh, pre_Wo, pre_bo)

    a1t_lo, a1t_hi, a1i_lo, a1i_hi = _agg128(
        y0i_lo, y0i_hi, src_it, dst_it, y0t_lo, y0t_hi, src_ti, dst_ti,
        zrow64)

    y1i_lo, y1i_hi, y1t_lo, y1t_hi = _l1(
        a1t_lo, a1t_hi, a1i_lo, a1i_hi, si_t, si_i, so_it, so_ti,
        c1_W_i2t, c1_b_i2t, c1_W_t2i, c1_b_t2i)

    a2t_lo, a2t_hi, a2i_lo, a2i_hi = _agg128(
        y1i_lo, y1i_hi, src_it, dst_it, y1t_lo, y1t_hi, src_ti, dst_ti,
        zrow64)

    z_lo, z_hi, z_ti = _l2(
        a2t_lo, a2t_hi, a2i_lo, a2i_hi, si_t, si_i, so_it, so_ti,
        c2_W_i2t, c2_b_i2t, c2_W_t2i, c2_b_t2i, c3_W_i2t, c3_W_t2i)

    a3_lo, a3_hi, a3e_a, a3e_b = _agg_l3(
        z_lo, z_hi, src_it, dst_it, z_ti.reshape(N), src_ti, dst_ti,
        zrow32, zvec)

    h3_t, h3_i = _post(a3_lo, a3_hi, a3e_a.reshape(N, 1), a3e_b.reshape(N, 1),
                       si_t, si_i, c3_b_i2t, c3_b_t2i,
                       post_Wi, post_bi, post_Wh, post_bh, post_Wo, post_bo)
    return (h3_t, h3_i)


# speedup vs baseline: 1.0004x; 1.0004x over previous
import functools

import jax
import jax.numpy as jnp
from jax import lax
from jax.experimental import pallas as pl
from jax.experimental.pallas import tpu as pltpu
from jax.experimental.pallas import tpu_sc as plsc

N = 10000
E = 320000
NT = 16
E_T = E // NT
CH = 80
NCH = E_T // CH
RT = 640
ACC_R = RT * NT
RT_LAST = N - RT * (NT - 1)

_MESH = plsc.VectorSubcoreMesh(core_axis_name="c", subcore_axis_name="s")
_f32 = jnp.float32
R = 5
NBLK = 5
BCH = NCH // NBLK


def _pipe_gather_scatter(xs, s4, d4, accs, sid, sidxb, didxb, lane_rows,
                         lane_gsems, lane_ssems):
    NL = len(xs)
    for blk in range(NBLK):
        pltpu.sync_copy(s4.at[sid * NBLK + blk], sidxb)
        pltpu.sync_copy(d4.at[sid * NBLK + blk], didxb)
        for L in range(NL):
            pltpu.async_copy(xs[L].at[sidxb.at[0]], lane_rows[L][0],
                             lane_gsems[L][0])

        def round_body(r, carry):
            for k in range(2):
                b = k
                bo = 1 - k
                c = r * 2 + k
                for L in range(NL):
                    pltpu.make_async_copy(xs[L].at[sidxb.at[0]],
                                          lane_rows[L][b],
                                          lane_gsems[L][b]).wait()
                for L in range(NL):
                    if k == 0:
                        @pl.when(r > 0)
                        def _(L=L):
                            pltpu.make_async_copy(
                                lane_rows[L][bo], accs[L].at[didxb.at[0]],
                                lane_ssems[L][bo]).wait()
                    else:
                        pltpu.make_async_copy(
                            lane_rows[L][bo], accs[L].at[didxb.at[0]],
                            lane_ssems[L][bo]).wait()

                for L in range(NL):
                    @pl.when(c + 1 < BCH)
                    def _(c=c, bo=bo, L=L):
                        pltpu.async_copy(xs[L].at[sidxb.at[c + 1]],
                                         lane_rows[L][bo], lane_gsems[L][bo])

                for L in range(NL):
                    pltpu.async_copy(lane_rows[L][b], accs[L].at[didxb.at[c]],
                                     lane_ssems[L][b], add=True)
            return carry

        lax.fori_loop(0, BCH // 2, round_body, 0)
        for L in range(NL):
            pltpu.make_async_copy(lane_rows[L][1], accs[L].at[didxb.at[0]],
                                  lane_ssems[L][1]).wait()


def _pipe_gather_scatter_par(x, s4, d4, accs2, sid, sidxb, didxb, rows4,
                             gsems4, ssems4):
    NB4 = (BCH // 4) * 4
    for blk in range(NBLK):
        pltpu.sync_copy(s4.at[sid * NBLK + blk], sidxb)
        pltpu.sync_copy(d4.at[sid * NBLK + blk], didxb)
        pltpu.async_copy(x.at[sidxb.at[0]], rows4[0], gsems4[0])
        pltpu.async_copy(x.at[sidxb.at[1]], rows4[1], gsems4[1])

        def round_body(r, carry):
            for k in range(4):
                b = k
                b2 = (k + 2) % 4
                c = r * 4 + k
                pltpu.make_async_copy(x.at[sidxb.at[0]], rows4[b],
                                      gsems4[b]).wait()
                if k < 2:
                    @pl.when(r > 0)
                    def _(b2=b2):
                        pltpu.make_async_copy(rows4[b2],
                                              accs2[b2 % 2].at[didxb.at[0]],
                                              ssems4[b2]).wait()
                else:
                    pltpu.make_async_copy(rows4[b2],
                                          accs2[b2 % 2].at[didxb.at[0]],
                                          ssems4[b2]).wait()
                pltpu.async_copy(x.at[sidxb.at[c + 2]], rows4[b2],
                                 gsems4[b2])
                pltpu.async_copy(rows4[b], accs2[b % 2].at[didxb.at[c]],
                                 ssems4[b], add=True)
            return carry

        lax.fori_loop(0, NB4 // 4, round_body, 0)
        for c in range(NB4, BCH):
            b = c % 4
            b2 = (c + 2) % 4
            pltpu.make_async_copy(x.at[sidxb.at[0]], rows4[b],
                                  gsems4[b]).wait()
            pltpu.make_async_copy(rows4[b2], accs2[b2 % 2].at[didxb.at[0]],
                                  ssems4[b2]).wait()
            pltpu.async_copy(rows4[b], accs2[b % 2].at[didxb.at[c]],
                             ssems4[b], add=True)
        for c in range(BCH - 2, BCH):
            b = c % 4
            pltpu.make_async_copy(rows4[b], accs2[b % 2].at[didxb.at[0]],
                                  ssems4[b]).wait()


def _pipe_hist(arr3, hists, sid, idx_all, onesv, ssems):
    pltpu.sync_copy(arr3.at[sid], idx_all)

    def round_body(r, carry):
        for k in range(R):
            c = r * R + k

            @pl.when(r > 0)
            def _(k=k):
                pltpu.make_async_copy(onesv, hists[k].at[idx_all.at[0]],
                                      ssems[k]).wait()

            pltpu.async_copy(onesv, hists[k].at[idx_all.at[c]], ssems[k],
                             add=True)
        return carry

    lax.fori_loop(0, NCH // R, round_body, 0)
    for k in range(R):
        pltpu.make_async_copy(onesv, hists[k].at[idx_all.at[0]],
                              ssems[k]).wait()


def _reduce_hists(hists, out, sid, rbuf, obuf, sz):
    off = sid * 640
    for j in range(R):
        pltpu.sync_copy(hists[j].at[pl.ds(off, sz)],
                        rbuf.at[j, pl.ds(0, sz)])
    for i in range(sz // 16):
        v = rbuf[0, pl.ds(16 * i, 16)]
        for j in range(1, R):
            v = v + rbuf[j, pl.ds(16 * i, 16)]
        obuf[pl.ds(16 * i, 16)] = v
    pltpu.sync_copy(obuf.at[pl.ds(0, sz)], out.at[pl.ds(off, sz)])


def _deg_kernel(src0, dst0, src1, dst1, ones_hbm, zvec):
    @functools.partial(
        pl.kernel,
        out_type=[jax.ShapeDtypeStruct((N,), _f32) for _ in range(4)],
        mesh=_MESH,
        scratch_types=[
            pltpu.VMEM((NCH, CH), jnp.int32),
            pltpu.VMEM((CH,), _f32),
            pltpu.VMEM((R, 640), _f32),
            pltpu.VMEM((640,), _f32),
        ] + [pltpu.VMEM_SHARED((N,), _f32)] * (2 * R)
          + [pltpu.SemaphoreType.DMA] * R,
        compiler_params=pltpu.CompilerParams(use_tc_tiling_on_sc=False),
    )
    def k(s0, d0, s1, d1, ones_h, zv, o0, o1, o2, o3, idx_all, onesv,
          rbuf, obuf, *bufs):
        hista = bufs[:R]
        histb = bufs[R:2 * R]
        ssems = bufs[2 * R:]
        sid = lax.axis_index("s")
        cid = lax.axis_index("c")
        pltpu.sync_copy(ones_h, onesv)

        @pl.when(sid == 0)
        def _():
            for h in hista + histb:
                pltpu.sync_copy(zv, h)

        plsc.subcore_barrier()

        @pl.when(cid == 0)
        def _():
            _pipe_hist(s0, hista, sid, idx_all, onesv, ssems)
            _pipe_hist(d0, histb, sid, idx_all, onesv, ssems)

        @pl.when(cid == 1)
        def _():
            _pipe_hist(s1, hista, sid, idx_all, onesv, ssems)
            _pipe_hist(d1, histb, sid, idx_all, onesv, ssems)

        plsc.subcore_barrier()

        for c, (oa, ob) in enumerate([(o0, o1), (o2, o3)]):
            @pl.when(jnp.logical_and(cid == c, sid < NT - 1))
            def _(oa=oa, ob=ob):
                _reduce_hists(hista, oa, sid, rbuf, obuf, 640)
                _reduce_hists(histb, ob, sid, rbuf, obuf, 640)

            @pl.when(jnp.logical_and(cid == c, sid == NT - 1))
            def _(oa=oa, ob=ob):
                _reduce_hists(hista, oa, sid, rbuf, obuf, 400)
                _reduce_hists(histb, ob, sid, rbuf, obuf, 400)

    return k(src0, dst0, src1, dst1, ones_hbm, zvec)


def _make_agg(D):
    H = D // 2

    @functools.partial(
        pl.kernel,
        out_type=[jax.ShapeDtypeStruct((N, H), _f32) for _ in range(4)],
        mesh=_MESH,
        scratch_types=[
            pltpu.VMEM((BCH, CH), jnp.int32),
            pltpu.VMEM((BCH, CH), jnp.int32),
            pltpu.VMEM_SHARED((ACC_R, H), _f32),
            pltpu.VMEM_SHARED((ACC_R, H), _f32),
        ] + [pltpu.VMEM((CH, H), _f32)] * 4
          + [pltpu.SemaphoreType.DMA] * 8,
        compiler_params=pltpu.CompilerParams(use_tc_tiling_on_sc=False),
    )
    def k(x0_lo, x0_hi, s0, d0, x1_lo, x1_hi, s1, d1, zrow,
          o0_lo, o0_hi, o1_lo, o1_hi, sidx_all, didx_all,
          acc_lo, acc_hi, *bufs):
        lane_rows = [bufs[0:2], bufs[2:4]]
        lane_gsems = [bufs[4:6], bufs[6:8]]
        lane_ssems = [bufs[8:10], bufs[10:12]]
        accs = (acc_lo, acc_hi)
        sid = lax.axis_index("s")
        cid = lax.axis_index("c")
        pltpu.sync_copy(zrow, acc_lo.at[pl.ds(sid * RT, RT)])
        pltpu.sync_copy(zrow, acc_hi.at[pl.ds(sid * RT, RT)])
        plsc.subcore_barrier()

        @pl.when(cid == 0)
        def _():
            _pipe_gather_scatter((x0_lo, x0_hi), s0, d0, accs, sid,
                                 sidx_all, didx_all, lane_rows,
                                 lane_gsems, lane_ssems)

        @pl.when(cid == 1)
        def _():
            _pipe_gather_scatter((x1_lo, x1_hi), s1, d1, accs, sid,
                                 sidx_all, didx_all, lane_rows,
                                 lane_gsems, lane_ssems)

        plsc.subcore_barrier()

        outs = [(o0_lo, o0_hi), (o1_lo, o1_hi)]
        for c in range(2):
            @pl.when(jnp.logical_and(cid == c, sid < NT - 1))
            def _(c=c):
                sl = pl.ds(sid * RT, RT)
                pltpu.sync_copy(acc_lo.at[sl], outs[c][0].at[sl])
                pltpu.sync_copy(acc_hi.at[sl], outs[c][1].at[sl])

            @pl.when(jnp.logical_and(cid == c, sid == NT - 1))
            def _(c=c):
                sl = pl.ds((NT - 1) * RT, RT_LAST)
                pltpu.sync_copy(acc_lo.at[sl], outs[c][0].at[sl])
                pltpu.sync_copy(acc_hi.at[sl], outs[c][1].at[sl])

    return k


_agg128 = _make_agg(128)


def _agg_l3(x32_lo, x32_hi, s0, d0, xe, s1, d1, zrow32, zvec):
    @functools.partial(
        pl.kernel,
        out_type=[jax.ShapeDtypeStruct((N, 32), _f32),
                  jax.ShapeDtypeStruct((N, 32), _f32),
                  jax.ShapeDtypeStruct((N,), _f32),
                  jax.ShapeDtypeStruct((N,), _f32)],
        mesh=_MESH,
        scratch_types=[
            pltpu.VMEM((BCH, CH), jnp.int32),
            pltpu.VMEM((BCH, CH), jnp.int32),
            pltpu.VMEM_SHARED((ACC_R, 32), _f32),
            pltpu.VMEM_SHARED((ACC_R, 32), _f32),
            pltpu.VMEM_SHARED((N,), _f32),
            pltpu.VMEM_SHARED((N,), _f32),
        ] + [pltpu.VMEM((CH, 32), _f32)] * 4
          + [pltpu.VMEM((CH,), _f32)] * 4
          + [pltpu.SemaphoreType.DMA] * 8,
        compiler_params=pltpu.CompilerParams(use_tc_tiling_on_sc=False),
    )
    def k(x_lo, x_hi, s_0, d_0, x_e, s_1, d_1, zr, zv,
          o32_lo, o32_hi, oe_a, oe_b, sidx_all, didx_all,
          acc_lo, acc_hi, acc1a, acc1b, *bufs):
        lane_rows = [bufs[0:2], bufs[2:4]]
        vals4 = bufs[4:8]
        sems_a = bufs[8:12]
        sems_b = bufs[12:16]
        sid = lax.axis_index("s")
        cid = lax.axis_index("c")

        @pl.when(cid == 0)
        def _():
            pltpu.sync_copy(zr, acc_lo.at[pl.ds(sid * RT, RT)])
            pltpu.sync_copy(zr, acc_hi.at[pl.ds(sid * RT, RT)])

        @pl.when(jnp.logical_and(cid == 1, sid == 0))
        def _():
            pltpu.sync_copy(zv, acc1a)
            pltpu.sync_copy(zv, acc1b)

        plsc.subcore_barrier()

        @pl.when(cid == 0)
        def _():
            _pipe_gather_scatter((x_lo, x_hi), s_0, d_0, (acc_lo, acc_hi),
                                 sid, sidx_all, didx_all, lane_rows,
                                 [sems_a[0:2], sems_a[2:4]],
                                 [sems_b[0:2], sems_b[2:4]])

        @pl.when(cid == 1)
        def _():
            _pipe_gather_scatter((x_e,), s_1, d_1, (acc1a,), sid,
                                 sidx_all, didx_all, [vals4[0:2]],
                                 [sems_a[0:2]], [sems_b[0:2]])

        plsc.subcore_barrier()

        @pl.when(jnp.logical_and(cid == 0, sid < NT - 1))
        def _():
            sl = pl.ds(sid * RT, RT)
            pltpu.sync_copy(acc_lo.at[sl], o32_lo.at[sl])
            pltpu.sync_copy(acc_hi.at[sl], o32_hi.at[sl])

        @pl.when(jnp.logical_and(cid == 0, sid == NT - 1))
        def _():
            sl = pl.ds((NT - 1) * RT, RT_LAST)
            pltpu.sync_copy(acc_lo.at[sl], o32_lo.at[sl])
            pltpu.sync_copy(acc_hi.at[sl], o32_hi.at[sl])

        @pl.when(jnp.logical_and(cid == 1, sid == 0))
        def _():
            pltpu.sync_copy(acc1a, oe_a)
            pltpu.sync_copy(acc1b, oe_b)

    return k(x32_lo, x32_hi, s0, d0, xe, s1, d1, zrow32, zvec)


_B = 1000
_G = N // _B


def _row_spec(d):
    return pl.BlockSpec((_B, d), lambda i: (i, 0))


def _full_spec(shape):
    if len(shape) == 1:
        return pl.BlockSpec(shape, lambda i: (0,))
    return pl.BlockSpec(shape, lambda i: (0, 0))


def _rs(d):
    return lax.rsqrt(jnp.maximum(d[...], 1.0))


def _pre_body(x, emb, so_ti, so_it, Wi, bi, Wh, bh, Wo, bo,
              y0t_lo, y0t_hi, y0i_lo, y0i_hi):
    h = jax.nn.relu(x[...] @ Wi[...] + bi[...])
    h = jax.nn.relu(h @ Wh[...] + bh[...])
    t = (h @ Wo[...] + bo[...]) * _rs(so_ti)
    y0t_lo[...] = t[:, :64]
    y0t_hi[...] = t[:, 64:]
    e = emb[...] * _rs(so_it)
    y0i_lo[...] = e[:, :64]
    y0i_hi[...] = e[:, 64:]


def _pre(x, emb, so_ti, so_it, Wi, bi, Wh, bh, Wo, bo):
    return pl.pallas_call(
        _pre_body,
        grid=(_G,),
        in_specs=[_row_spec(256), _row_spec(128), _row_spec(1), _row_spec(1),
                  _full_spec(Wi.shape), _full_spec(bi.shape),
                  _full_spec(Wh.shape), _full_spec(bh.shape),
                  _full_spec(Wo.shape), _full_spec(bo.shape)],
        out_specs=[_row_spec(64)] * 4,
        out_shape=[jax.ShapeDtypeStruct((N, 64), _f32)] * 4,
    )(x, emb, so_ti, so_it, Wi, bi, Wh, bh, Wo, bo)


def _l1_body(at_lo, at_hi, ai_lo, ai_hi, si_t, si_i, so_it, so_ti,
             W_it_a, W_it_b, b_it, W_ti_a, W_ti_b, b_ti,
             y_i_lo, y_i_hi, y_t_lo, y_t_hi):
    si = _rs(si_t)
    h_t = jax.nn.relu((at_lo[...] * si) @ W_it_a[...]
                      + (at_hi[...] * si) @ W_it_b[...] + b_it[...])
    y_t = h_t * _rs(so_ti)
    y_t_lo[...] = y_t[:, :64]
    y_t_hi[...] = y_t[:, 64:]
    si2 = _rs(si_i)
    h_i = jax.nn.relu((ai_lo[...] * si2) @ W_ti_a[...]
                      + (ai_hi[...] * si2) @ W_ti_b[...] + b_ti[...])
    y_i = h_i * _rs(so_it)
    y_i_lo[...] = y_i[:, :64]
    y_i_hi[...] = y_i[:, 64:]


def _l1(at_lo, at_hi, ai_lo, ai_hi, si_t, si_i, so_it, so_ti,
        W_it, b_it, W_ti, b_ti):
    return pl.pallas_call(
        _l1_body,
        grid=(_G,),
        in_specs=[_row_spec(64)] * 4 + [_row_spec(1)] * 4 +
                 [_full_spec((64, 128)), _full_spec((64, 128)),
                  _full_spec(b_it.shape),
                  _full_spec((64, 128)), _full_spec((64, 128)),
                  _full_spec(b_ti.shape)],
        out_specs=[_row_spec(64)] * 4,
        out_shape=[jax.ShapeDtypeStruct((N, 64), _f32)] * 4,
    )(at_lo, at_hi, ai_lo, ai_hi, si_t, si_i, so_it, so_ti,
      W_it[:64], W_it[64:], b_it, W_ti[:64], W_ti[64:], b_ti)


def _l2_body(at_lo, at_hi, ai_lo, ai_hi, si_t, si_i, so_it, so_ti,
             W_it_a, W_it_b, b_it, W_ti_a, W_ti_b, b_ti,
             W3_it_a, W3_it_b, W3_ti,
             z_lo, z_hi, z_ti):
    si = _rs(si_t)
    si2 = _rs(si_i)
    h2_t = jax.nn.relu((at_lo[...] * si) @ W_it_a[...]
                       + (at_hi[...] * si) @ W_it_b[...] + b_it[...])
    h2_i = jax.nn.relu((ai_lo[...] * si2) @ W_ti_a[...]
                       + (ai_hi[...] * si2) @ W_ti_b[...] + b_ti[...])
    zi = h2_i * _rs(so_it)
    z_lo[...] = zi @ W3_it_a[...]
    z_hi[...] = zi @ W3_it_b[...]
    zt = h2_t * _rs(so_ti)
    z_ti[...] = zt @ W3_ti[...]


def _l2(at_lo, at_hi, ai_lo, ai_hi, si_t, si_i, so_it, so_ti,
        W_it, b_it, W_ti, b_ti, W3_it, W3_ti):
    return pl.pallas_call(
        _l2_body,
        grid=(_G,),
        in_specs=[_row_spec(64)] * 4 + [_row_spec(1)] * 4 +
                 [_full_spec((64, 128)), _full_spec((64, 128)),
                  _full_spec(b_it.shape),
                  _full_spec((64, 128)), _full_spec((64, 128)),
                  _full_spec(b_ti.shape),
                  _full_spec((128, 32)), _full_spec((128, 32)),
                  _full_spec((128, 1))],
        out_specs=[_row_spec(32), _row_spec(32), _row_spec(1)],
        out_shape=[jax.ShapeDtypeStruct((N, 32), _f32),
                   jax.ShapeDtypeStruct((N, 32), _f32),
                   jax.ShapeDtypeStruct((N, 1), _f32)],
    )(at_lo, at_hi, ai_lo, ai_hi, si_t, si_i, so_it, so_ti,
      W_it[:64], W_it[64:], b_it, W_ti[:64], W_ti[64:], b_ti,
      W3_it[:, :32], W3_it[:, 32:], W3_ti)


def _post_body(a32_lo, a32_hi, age_a, age_b, si_t, si_i,
               b3_lo, b3_hi, b3_ti, Wi_a, Wi_b, bi, Wh, bh, Wo, bo,
               o_t, o_i):
    si = _rs(si_t)
    t_lo = a32_lo[...] * si + b3_lo[...]
    t_hi = a32_hi[...] * si + b3_hi[...]
    h = jax.nn.relu(t_lo @ Wi_a[...] + t_hi @ Wi_b[...] + bi[...])
    h = jax.nn.relu(h @ Wh[...] + bh[...])
    o_t[...] = h @ Wo[...] + bo[...]
    o_i[...] = (age_a[...] + age_b[...]) * _rs(si_i) + b3_ti[...]


def _post(a32_lo, a32_hi, age_a, age_b, si_t, si_i, b3_it, b3_ti,
          Wi, bi, Wh, bh, Wo, bo):
    return pl.pallas_call(
        _post_body,
        grid=(_G,),
        in_specs=[_row_spec(32), _row_spec(32), _row_spec(1), _row_spec(1),
                  _row_spec(1), _row_spec(1),
                  _full_spec((32,)), _full_spec((32,)),
                  _full_spec(b3_ti.shape),
                  _full_spec((32, 64)), _full_spec((32, 64)),
                  _full_spec(bi.shape),
                  _full_spec(Wh.shape), _full_spec(bh.shape),
                  _full_spec(Wo.shape), _full_spec(bo.shape)],
        out_specs=[_row_spec(1), _row_spec(1)],
        out_shape=[jax.ShapeDtypeStruct((N, 1), _f32),
                   jax.ShapeDtypeStruct((N, 1), _f32)],
    )(a32_lo, a32_hi, age_a, age_b, si_t, si_i, b3_it[:32], b3_it[32:],
      b3_ti, Wi[:32], Wi[32:], bi, Wh, bh, Wo, bo)


def kernel(input_features, edge_i2t, edge_t2i, embed_item,
           pre_Wi, pre_bi, pre_Wh, pre_bh, pre_Wo, pre_bo,
           c1_W_i2t, c1_b_i2t, c1_W_t2i, c1_b_t2i,
           c2_W_i2t, c2_b_i2t, c2_W_t2i, c2_b_t2i,
           c3_W_i2t, c3_b_i2t, c3_W_t2i, c3_b_t2i,
           post_Wi, post_bi, post_Wh, post_bh, post_Wo, post_bo):
    src_it = edge_i2t[0].reshape(NT * NBLK, BCH, CH)
    dst_it = edge_i2t[1].reshape(NT * NBLK, BCH, CH)
    src_ti = edge_t2i[0].reshape(NT * NBLK, BCH, CH)
    dst_ti = edge_t2i[1].reshape(NT * NBLK, BCH, CH)
    src_it_t = edge_i2t[0].reshape(NT, NCH, CH)
    dst_it_t = edge_i2t[1].reshape(NT, NCH, CH)
    src_ti_t = edge_t2i[0].reshape(NT, NCH, CH)
    dst_ti_t = edge_t2i[1].reshape(NT, NCH, CH)

    ones_hbm = jnp.ones((CH,), _f32)
    zvec = jnp.zeros((N,), _f32)
    zrow64 = jnp.zeros((RT, 64), _f32)
    zrow32 = jnp.zeros((RT, 32), _f32)

    d_out_it, d_in_t, d_out_ti, d_in_i = _deg_kernel(
        src_it_t, dst_it_t, src_ti_t, dst_ti_t, ones_hbm, zvec)

    so_it, si_t, so_ti, si_i = (d_out_it.reshape(N, 1), d_in_t.reshape(N, 1),
                                d_out_ti.reshape(N, 1), d_in_i.reshape(N, 1))

    y0t_lo, y0t_hi, y0i_lo, y0i_hi = _pre(
        input_features.astype(_f32), embed_item, so_ti, so_it,
        pre_Wi, pre_bi, pre_Wh, pre_bh, pre_Wo, pre_bo)

    a1t_lo, a1t_hi, a1i_lo, a1i_hi = _agg128(
        y0i_lo, y0i_hi, src_it, dst_it, y0t_lo, y0t_hi, src_ti, dst_ti,
        zrow64)

    y1i_lo, y1i_hi, y1t_lo, y1t_hi = _l1(
        a1t_lo, a1t_hi, a1i_lo, a1i_hi, si_t, si_i, so_it, so_ti,
        c1_W_i2t, c1_b_i2t, c1_W_t2i, c1_b_t2i)

    a2t_lo, a2t_hi, a2i_lo, a2i_hi = _agg128(
        y1i_lo, y1i_hi, src_it, dst_it, y1t_lo, y1t_hi, src_ti, dst_ti,
        zrow64)

    z_lo, z_hi, z_ti = _l2(
        a2t_lo, a2t_hi, a2i_lo, a2i_hi, si_t, si_i, so_it, so_ti,
        c2_W_i2t, c2_b_i2t, c2_W_t2i, c2_b_t2i, c3_W_i2t, c3_W_t2i)

    a3_lo, a3_hi, a3e_a, a3e_b = _agg_l3(
        z_lo, z_hi, src_it, dst_it, z_ti.reshape(N), src_ti, dst_ti,
        zrow32, zvec)

    h3_t, h3_i = _post(a3_lo, a3_hi, a3e_a.reshape(N, 1), a3e_b.reshape(N, 1),
                       si_t, si_i, c3_b_i2t, c3_b_t2i,
                       post_Wi, post_bi, post_Wh, post_bh, post_Wo, post_bo)
    return (h3_t, h3_i)
